# Initial kernel scaffold; baseline (speedup 1.0000x reference)
import jax
import jax.numpy as jnp
from jax.experimental import pallas as pl


def kernel(latent, edge_index_list, pos_list_scale, pos_list, W_dec0, b_dec0, Wk0, bk0, Wk1, bk1, Wk2, bk2, conv_bias, W_dec1, b_dec1):
    raise NotImplementedError("write your pallas kernel here")


# speedup vs baseline: 1.0748x; 1.0748x over previous
import functools
import jax
import jax.numpy as jnp
from jax.experimental import pallas as pl
from jax.experimental.pallas import tpu as pltpu


def _node_body(latent_ref, pos_ref, Wd0_ref, bd0_ref, Wk2x_ref, Wk2p_ref,
               bk2x_ref, bk2p_ref, z_ref, s_ref):
    x = jnp.sin(0.01 * (jnp.dot(latent_ref[...], Wd0_ref[...],
                                preferred_element_type=jnp.float32) + bd0_ref[...]))
    posb = pos_ref[...]
    z_ref[...] = (jnp.dot(x, Wk2x_ref[...], preferred_element_type=jnp.float32)
                  + jnp.dot(posb, Wk2p_ref[...], preferred_element_type=jnp.float32))
    s_ref[...] = (jnp.dot(x, bk2x_ref[...], preferred_element_type=jnp.float32)
                  + jnp.dot(posb, bk2p_ref[...], preferred_element_type=jnp.float32))


def _node_stage(latent, pos, W_dec0, b_dec0, Wk2, bk2, nb):
    n, l = latent.shape
    h = Wk2.shape[0]
    d = pos.shape[1]
    grid = n // nb
    full = lambda i: (0, 0)
    z, s = pl.pallas_call(
        _node_body,
        grid=(grid,),
        in_specs=[
            pl.BlockSpec((nb, l), lambda i: (i, 0)),
            pl.BlockSpec((nb, d), lambda i: (i, 0)),
            pl.BlockSpec((l, h), full),
            pl.BlockSpec((1, h), full),
            pl.BlockSpec((h, h), full),
            pl.BlockSpec((d, h), full),
            pl.BlockSpec((h, 1), full),
            pl.BlockSpec((d, 1), full),
        ],
        out_specs=[
            pl.BlockSpec((nb, h), lambda i: (i, 0)),
            pl.BlockSpec((nb, 1), lambda i: (i, 0)),
        ],
        out_shape=[
            jax.ShapeDtypeStruct((n, h), jnp.float32),
            jax.ShapeDtypeStruct((n, 1), jnp.float32),
        ],
    )(latent, pos, W_dec0, b_dec0[None, :], Wk2[:, :h].T, Wk2[:, h:].T,
      bk2[:h, None], bk2[h:, None])
    return z, s


def _edge_body(sb_ref, cb_ref, y_ref, se_ref, cS_ref, sS_ref, W1_ref, b1_ref,
               msg_ref, *, h):
    sb = sb_ref[...]
    cb = cb_ref[...]
    y = y_ref[...]
    cols = []
    for c in range(h):
        h0 = sb * cS_ref[c:c + 1, :] + cb * sS_ref[c:c + 1, :]
        g = jnp.dot(h0, W1_ref[...], preferred_element_type=jnp.float32) + b1_ref[...]
        h1 = jnp.sin(g)
        cols.append(jnp.sum(h1 * y, axis=1, keepdims=True))
    msg_ref[...] = jnp.concatenate(cols, axis=1) + se_ref[...]


def _edge_stage(sb, cb, y, se, cS, sS, Wk1s, bk1s, eb):
    e, h = sb.shape
    grid = e // eb
    full = lambda i: (0, 0)
    return pl.pallas_call(
        functools.partial(_edge_body, h=h),
        grid=(grid,),
        in_specs=[
            pl.BlockSpec((eb, h), lambda i: (i, 0)),
            pl.BlockSpec((eb, h), lambda i: (i, 0)),
            pl.BlockSpec((eb, h), lambda i: (i, 0)),
            pl.BlockSpec((eb, 1), lambda i: (i, 0)),
            pl.BlockSpec((h, h), full),
            pl.BlockSpec((h, h), full),
            pl.BlockSpec((h, h), full),
            pl.BlockSpec((1, h), full),
        ],
        out_specs=pl.BlockSpec((eb, h), lambda i: (i, 0)),
        out_shape=jax.ShapeDtypeStruct((e, h), jnp.float32),
    )(sb, cb, y, se, cS, sS, Wk1s, bk1s)


def _final_body(agg_ref, z_ref, s_ref, H1s_ref, cb_ref, Wd1_ref, bd1_ref, out_ref):
    selfterm = jnp.dot(z_ref[...], H1s_ref[...], preferred_element_type=jnp.float32)
    x2 = jnp.sin(0.01 * (agg_ref[...] + selfterm + s_ref[...] + cb_ref[...]))
    out_ref[...] = jnp.dot(x2, Wd1_ref[...], preferred_element_type=jnp.float32) + bd1_ref[...]


def _final_stage(agg, z, s, H1self, conv_bias, W_dec1, b_dec1, nb):
    n, h = agg.shape
    out_d = W_dec1.shape[1]
    grid = n // nb
    full = lambda i: (0, 0)
    return pl.pallas_call(
        _final_body,
        grid=(grid,),
        in_specs=[
            pl.BlockSpec((nb, h), lambda i: (i, 0)),
            pl.BlockSpec((nb, h), lambda i: (i, 0)),
            pl.BlockSpec((nb, 1), lambda i: (i, 0)),
            pl.BlockSpec((h, h), full),
            pl.BlockSpec((1, h), full),
            pl.BlockSpec((h, out_d), full),
            pl.BlockSpec((1, out_d), full),
        ],
        out_specs=pl.BlockSpec((nb, out_d), lambda i: (i, 0)),
        out_shape=jax.ShapeDtypeStruct((n, out_d), jnp.float32),
    )(agg, z, s, H1self.T, conv_bias, W_dec1, b_dec1[None, :])


def kernel(latent, edge_index_list, pos_list_scale, pos_list, W_dec0, b_dec0,
           Wk0, bk0, Wk1, bk1, Wk2, bk2, conv_bias, W_dec1, b_dec1):
    omega = 0.01
    n, l = latent.shape
    h = W_dec0.shape[1]
    pos = pos_list_scale[0]
    d = pos.shape[1]
    edge_index = edge_index_list[0]
    src, dst = edge_index[0], edge_index[1]

    cvec = jnp.arange(h, dtype=jnp.float32)
    shift = 0.1 * cvec[:, None] * Wk0[d][None, :]
    cS, sS = jnp.cos(shift), jnp.sin(shift)
    h0s = jnp.sin(0.1 * bk0[None, :] + shift)
    H1self = jnp.sin(h0s @ (0.1 * Wk1) + 0.1 * bk1)
    Wk1s = 0.1 * Wk1
    bk1s = (0.1 * bk1)[None, :]

    z, s = _node_stage(latent, pos, W_dec0, b_dec0, Wk2, bk2, nb=2000)

    rel = pos[dst] - pos[src]
    sq = jnp.sum(rel * rel, axis=1)
    mask = sq > 0
    rho = jnp.where(mask, jnp.sqrt(jnp.where(mask, sq, 1.0)), 0.0)
    theta = jnp.arctan2(jnp.where(mask, rel[:, 1], 0.0),
                        jnp.where(mask, rel[:, 0], 1.0))
    ratio = jnp.where(mask, rel[:, 2] / jnp.where(mask, rho, 1.0), 0.0)
    phi = jnp.arcsin(jnp.clip(ratio, -1.0, 1.0))
    theta = jnp.where(mask, theta, 0.0)
    phi = jnp.where(mask, phi, 0.0)
    feats = jnp.stack([rho, theta / jnp.pi, phi / jnp.pi], axis=1)

    base = 0.1 * (feats @ Wk0[:d] + bk0)
    sb, cb = jnp.sin(base), jnp.cos(base)
    y = z[src]
    se = s[src]

    msg = _edge_stage(sb, cb, y, se, cS, sS, Wk1s, bk1s, eb=2000)

    agg = jax.ops.segment_sum(msg, dst, num_segments=n)

    return _final_stage(agg, z, s, H1self, conv_bias, W_dec1, b_dec1, nb=2000)


# speedup vs baseline: 4.1267x; 3.8395x over previous
import functools
import jax
import jax.numpy as jnp
from jax.experimental import pallas as pl
from jax.experimental.pallas import tpu as pltpu


def _node_body(latent_ref, pos_ref, Wd0_ref, bd0_ref, Wk2x_ref, Wk2p_ref,
               bk2x_ref, bk2p_ref, z_ref, s_ref):
    x = jnp.sin(0.01 * (jnp.dot(latent_ref[...], Wd0_ref[...],
                                preferred_element_type=jnp.float32) + bd0_ref[...]))
    posb = pos_ref[...]
    z_ref[...] = (jnp.dot(x, Wk2x_ref[...], preferred_element_type=jnp.float32)
                  + jnp.dot(posb, Wk2p_ref[...], preferred_element_type=jnp.float32))
    s_ref[...] = (jnp.dot(x, bk2x_ref[...], preferred_element_type=jnp.float32)
                  + jnp.dot(posb, bk2p_ref[...], preferred_element_type=jnp.float32))


def _node_stage(latent, pos, W_dec0, b_dec0, Wk2, bk2, nb):
    n, l = latent.shape
    h = Wk2.shape[0]
    d = pos.shape[1]
    grid = n // nb
    full = lambda i: (0, 0)
    z, s = pl.pallas_call(
        _node_body,
        grid=(grid,),
        in_specs=[
            pl.BlockSpec((nb, l), lambda i: (i, 0)),
            pl.BlockSpec((nb, d), lambda i: (i, 0)),
            pl.BlockSpec((l, h), full),
            pl.BlockSpec((1, h), full),
            pl.BlockSpec((h, h), full),
            pl.BlockSpec((d, h), full),
            pl.BlockSpec((h, 1), full),
            pl.BlockSpec((d, 1), full),
        ],
        out_specs=[
            pl.BlockSpec((nb, h), lambda i: (i, 0)),
            pl.BlockSpec((nb, 1), lambda i: (i, 0)),
        ],
        out_shape=[
            jax.ShapeDtypeStruct((n, h), jnp.float32),
            jax.ShapeDtypeStruct((n, 1), jnp.float32),
        ],
    )(latent, pos, W_dec0, b_dec0[None, :], Wk2[:, :h].T, Wk2[:, h:].T,
      bk2[:h, None], bk2[h:, None])
    return z, s


_S3 = -1.0 / 6.0
_S5 = 1.0 / 120.0
_S7 = -1.0 / 5040.0
_S9 = 1.0 / 362880.0


def _sin_poly(x):
    x2 = x * x
    return ((((_S9 * x2 + _S7) * x2 + _S5) * x2 + _S3) * x2 + 1.0) * x


def _edge_body(sb_ref, cb_ref, y_ref, se_ref, cSg_ref, sSg_ref, W1d_ref,
               b1d_ref, msg_ref, *, ngrp):
    sb = sb_ref[...]
    cb = cb_ref[...]
    y = y_ref[...]
    sb4 = jnp.concatenate([sb] * 4, axis=1)
    cb4 = jnp.concatenate([cb] * 4, axis=1)
    y4 = jnp.concatenate([y] * 4, axis=1)
    cols = []
    for g in range(ngrp):
        h0 = sb4 * cSg_ref[g:g + 1, :] + cb4 * sSg_ref[g:g + 1, :]
        gm = jnp.dot(h0, W1d_ref[...],
                     preferred_element_type=jnp.float32) + b1d_ref[...]
        p = _sin_poly(gm) * y4
        for ci in range(4):
            cols.append(jnp.sum(p[:, ci * 64:(ci + 1) * 64], axis=1,
                                keepdims=True))
    msg_ref[...] = jnp.concatenate(cols, axis=1) + se_ref[...]


def _edge_stage(sb, cb, y, se, cSg, sSg, W1d, b1d, eb):
    e, h = sb.shape
    ngrp = h // 4
    grid = e // eb
    full = lambda i: (0, 0)
    return pl.pallas_call(
        functools.partial(_edge_body, ngrp=ngrp),
        grid=(grid,),
        in_specs=[
            pl.BlockSpec((eb, h), lambda i: (i, 0)),
            pl.BlockSpec((eb, h), lambda i: (i, 0)),
            pl.BlockSpec((eb, h), lambda i: (i, 0)),
            pl.BlockSpec((eb, 1), lambda i: (i, 0)),
            pl.BlockSpec((ngrp, 4 * h), full),
            pl.BlockSpec((ngrp, 4 * h), full),
            pl.BlockSpec((4 * h, 4 * h), full),
            pl.BlockSpec((1, 4 * h), full),
        ],
        out_specs=pl.BlockSpec((eb, h), lambda i: (i, 0)),
        out_shape=jax.ShapeDtypeStruct((e, h), jnp.float32),
    )(sb, cb, y, se, cSg, sSg, W1d, b1d)


def _final_body(agg_ref, z_ref, s_ref, H1s_ref, cb_ref, Wd1_ref, bd1_ref, out_ref):
    selfterm = jnp.dot(z_ref[...], H1s_ref[...], preferred_element_type=jnp.float32)
    x2 = jnp.sin(0.01 * (agg_ref[...] + selfterm + s_ref[...] + cb_ref[...]))
    out_ref[...] = jnp.dot(x2, Wd1_ref[...], preferred_element_type=jnp.float32) + bd1_ref[...]


def _final_stage(agg, z, s, H1self, conv_bias, W_dec1, b_dec1, nb):
    n, h = agg.shape
    out_d = W_dec1.shape[1]
    grid = n // nb
    full = lambda i: (0, 0)
    return pl.pallas_call(
        _final_body,
        grid=(grid,),
        in_specs=[
            pl.BlockSpec((nb, h), lambda i: (i, 0)),
            pl.BlockSpec((nb, h), lambda i: (i, 0)),
            pl.BlockSpec((nb, 1), lambda i: (i, 0)),
            pl.BlockSpec((h, h), full),
            pl.BlockSpec((1, h), full),
            pl.BlockSpec((h, out_d), full),
            pl.BlockSpec((1, out_d), full),
        ],
        out_specs=pl.BlockSpec((nb, out_d), lambda i: (i, 0)),
        out_shape=jax.ShapeDtypeStruct((n, out_d), jnp.float32),
    )(agg, z, s, H1self.T, conv_bias, W_dec1, b_dec1[None, :])


def kernel(latent, edge_index_list, pos_list_scale, pos_list, W_dec0, b_dec0,
           Wk0, bk0, Wk1, bk1, Wk2, bk2, conv_bias, W_dec1, b_dec1):
    omega = 0.01
    n, l = latent.shape
    h = W_dec0.shape[1]
    pos = pos_list_scale[0]
    d = pos.shape[1]
    edge_index = edge_index_list[0]
    src, dst = edge_index[0], edge_index[1]

    cvec = jnp.arange(h, dtype=jnp.float32)
    shift = 0.1 * cvec[:, None] * Wk0[d][None, :]
    cS, sS = jnp.cos(shift), jnp.sin(shift)
    h0s = jnp.sin(0.1 * bk0[None, :] + shift)
    H1self = jnp.sin(h0s @ (0.1 * Wk1) + 0.1 * bk1)

    ngrp = h // 4
    cSg = cS.reshape(ngrp, 4 * h)
    sSg = sS.reshape(ngrp, 4 * h)
    W1d = jax.scipy.linalg.block_diag(*([0.1 * Wk1] * 4))
    b1d = jnp.tile(0.1 * bk1, 4)[None, :]

    z, s = _node_stage(latent, pos, W_dec0, b_dec0, Wk2, bk2, nb=2000)

    rel = pos[dst] - pos[src]
    sq = jnp.sum(rel * rel, axis=1)
    mask = sq > 0
    rho = jnp.where(mask, jnp.sqrt(jnp.where(mask, sq, 1.0)), 0.0)
    theta = jnp.arctan2(jnp.where(mask, rel[:, 1], 0.0),
                        jnp.where(mask, rel[:, 0], 1.0))
    ratio = jnp.where(mask, rel[:, 2] / jnp.where(mask, rho, 1.0), 0.0)
    phi = jnp.arcsin(jnp.clip(ratio, -1.0, 1.0))
    theta = jnp.where(mask, theta, 0.0)
    phi = jnp.where(mask, phi, 0.0)
    feats = jnp.stack([rho, theta / jnp.pi, phi / jnp.pi], axis=1)

    base = 0.1 * (feats @ Wk0[:d] + bk0)
    sb, cb = jnp.sin(base), jnp.cos(base)
    y = z[src]
    se = s[src]

    msg = _edge_stage(sb, cb, y, se, cSg, sSg, W1d, b1d, eb=2000)

    agg = jax.ops.segment_sum(msg, dst, num_segments=n)

    return _final_stage(agg, z, s, H1self, conv_bias, W_dec1, b_dec1, nb=2000)


# speedup vs baseline: 5.7347x; 1.3897x over previous
import functools
import jax
import jax.numpy as jnp
from jax.experimental import pallas as pl
from jax.experimental.pallas import tpu as pltpu


def _node_body(latent_ref, pos_ref, Wd0_ref, bd0_ref, Wk2x_ref, Wk2p_ref,
               bk2x_ref, bk2p_ref, z_ref, s_ref):
    x = jnp.sin(0.01 * (jnp.dot(latent_ref[...], Wd0_ref[...],
                                preferred_element_type=jnp.float32) + bd0_ref[...]))
    posb = pos_ref[...]
    z_ref[...] = (jnp.dot(x, Wk2x_ref[...], preferred_element_type=jnp.float32)
                  + jnp.dot(posb, Wk2p_ref[...], preferred_element_type=jnp.float32))
    s_ref[...] = (jnp.dot(x, bk2x_ref[...], preferred_element_type=jnp.float32)
                  + jnp.dot(posb, bk2p_ref[...], preferred_element_type=jnp.float32))


def _node_stage(latent, pos, W_dec0, b_dec0, Wk2, bk2, nb):
    n, l = latent.shape
    h = Wk2.shape[0]
    d = pos.shape[1]
    grid = n // nb
    full = lambda i: (0, 0)
    z, s = pl.pallas_call(
        _node_body,
        grid=(grid,),
        in_specs=[
            pl.BlockSpec((nb, l), lambda i: (i, 0)),
            pl.BlockSpec((nb, d), lambda i: (i, 0)),
            pl.BlockSpec((l, h), full),
            pl.BlockSpec((1, h), full),
            pl.BlockSpec((h, h), full),
            pl.BlockSpec((d, h), full),
            pl.BlockSpec((h, 1), full),
            pl.BlockSpec((d, 1), full),
        ],
        out_specs=[
            pl.BlockSpec((nb, h), lambda i: (i, 0)),
            pl.BlockSpec((nb, 1), lambda i: (i, 0)),
        ],
        out_shape=[
            jax.ShapeDtypeStruct((n, h), jnp.float32),
            jax.ShapeDtypeStruct((n, 1), jnp.float32),
        ],
    )(latent, pos, W_dec0, b_dec0[None, :], Wk2[:, :h].T, Wk2[:, h:].T,
      bk2[:h, None], bk2[h:, None])
    return z, s


_S3 = -1.0 / 6.0
_S5 = 1.0 / 120.0
_S7 = -1.0 / 5040.0
_S9 = 1.0 / 362880.0


def _sin_poly(x):
    x2 = x * x
    return ((((_S9 * x2 + _S7) * x2 + _S5) * x2 + _S3) * x2 + 1.0) * x


def _edge_body(feats_ref, y_ref, se_ref, W0p_ref, b0_ref, cSg_ref, sSg_ref,
               W1d_ref, b1d_ref, msg_ref, *, ngrp):
    base = jnp.dot(feats_ref[...], W0p_ref[...],
                   preferred_element_type=jnp.float32) + b0_ref[...]
    sb = jnp.sin(base)
    cb = jnp.cos(base)
    y = y_ref[...]
    sb4 = jnp.concatenate([sb] * 4, axis=1)
    cb4 = jnp.concatenate([cb] * 4, axis=1)
    y4 = jnp.concatenate([y] * 4, axis=1)
    cols = []
    for g in range(ngrp):
        h0 = sb4 * cSg_ref[g:g + 1, :] + cb4 * sSg_ref[g:g + 1, :]
        gm = jnp.dot(h0, W1d_ref[...],
                     preferred_element_type=jnp.float32) + b1d_ref[...]
        p = _sin_poly(gm) * y4
        for ci in range(4):
            cols.append(jnp.sum(p[:, ci * 64:(ci + 1) * 64], axis=1,
                                keepdims=True))
    msg_ref[...] = jnp.concatenate(cols, axis=1) + se_ref[...]


def _edge_stage(feats, y, se, W0p, b0, cSg, sSg, W1d, b1d, eb):
    e, h = y.shape
    fd = feats.shape[1]
    ngrp = h // 4
    grid = e // eb
    full = lambda i: (0, 0)
    return pl.pallas_call(
        functools.partial(_edge_body, ngrp=ngrp),
        grid=(grid,),
        in_specs=[
            pl.BlockSpec((eb, fd), lambda i: (i, 0)),
            pl.BlockSpec((eb, h), lambda i: (i, 0)),
            pl.BlockSpec((eb, 1), lambda i: (i, 0)),
            pl.BlockSpec((fd, h), full),
            pl.BlockSpec((1, h), full),
            pl.BlockSpec((ngrp, 4 * h), full),
            pl.BlockSpec((ngrp, 4 * h), full),
            pl.BlockSpec((4 * h, 4 * h), full),
            pl.BlockSpec((1, 4 * h), full),
        ],
        out_specs=pl.BlockSpec((eb, h), lambda i: (i, 0)),
        out_shape=jax.ShapeDtypeStruct((e, h), jnp.float32),
    )(feats, y, se, W0p, b0, cSg, sSg, W1d, b1d)


def _final_body(agg_ref, z_ref, s_ref, H1s_ref, cb_ref, Wd1_ref, bd1_ref, out_ref):
    selfterm = jnp.dot(z_ref[...], H1s_ref[...], preferred_element_type=jnp.float32)
    x2 = jnp.sin(0.01 * (agg_ref[...] + selfterm + s_ref[...] + cb_ref[...]))
    out_ref[...] = jnp.dot(x2, Wd1_ref[...], preferred_element_type=jnp.float32) + bd1_ref[...]


def _final_stage(agg, z, s, H1self, conv_bias, W_dec1, b_dec1, nb):
    n, h = agg.shape
    out_d = W_dec1.shape[1]
    grid = n // nb
    full = lambda i: (0, 0)
    return pl.pallas_call(
        _final_body,
        grid=(grid,),
        in_specs=[
            pl.BlockSpec((nb, h), lambda i: (i, 0)),
            pl.BlockSpec((nb, h), lambda i: (i, 0)),
            pl.BlockSpec((nb, 1), lambda i: (i, 0)),
            pl.BlockSpec((h, h), full),
            pl.BlockSpec((1, h), full),
            pl.BlockSpec((h, out_d), full),
            pl.BlockSpec((1, out_d), full),
        ],
        out_specs=pl.BlockSpec((nb, out_d), lambda i: (i, 0)),
        out_shape=jax.ShapeDtypeStruct((n, out_d), jnp.float32),
    )(agg, z, s, H1self.T, conv_bias, W_dec1, b_dec1[None, :])


def kernel(latent, edge_index_list, pos_list_scale, pos_list, W_dec0, b_dec0,
           Wk0, bk0, Wk1, bk1, Wk2, bk2, conv_bias, W_dec1, b_dec1):
    omega = 0.01
    n, l = latent.shape
    h = W_dec0.shape[1]
    pos = pos_list_scale[0]
    d = pos.shape[1]
    edge_index = edge_index_list[0]
    src, dst = edge_index[0], edge_index[1]

    cvec = jnp.arange(h, dtype=jnp.float32)
    shift = 0.1 * cvec[:, None] * Wk0[d][None, :]
    cS, sS = jnp.cos(shift), jnp.sin(shift)
    h0s = jnp.sin(0.1 * bk0[None, :] + shift)
    H1self = jnp.sin(h0s @ (0.1 * Wk1) + 0.1 * bk1)

    ngrp = h // 4
    cSg = cS.reshape(ngrp, 4 * h)
    sSg = sS.reshape(ngrp, 4 * h)
    W1d = jax.scipy.linalg.block_diag(*([0.1 * Wk1] * 4))
    b1d = jnp.tile(0.1 * bk1, 4)[None, :]

    z, s = _node_stage(latent, pos, W_dec0, b_dec0, Wk2, bk2, nb=2000)

    rel = pos[dst] - pos[src]
    sq = jnp.sum(rel * rel, axis=1)
    mask = sq > 0
    rho = jnp.where(mask, jnp.sqrt(jnp.where(mask, sq, 1.0)), 0.0)
    theta = jnp.arctan2(jnp.where(mask, rel[:, 1], 0.0),
                        jnp.where(mask, rel[:, 0], 1.0))
    ratio = jnp.where(mask, rel[:, 2] / jnp.where(mask, rho, 1.0), 0.0)
    phi = jnp.arcsin(jnp.clip(ratio, -1.0, 1.0))
    theta = jnp.where(mask, theta, 0.0)
    phi = jnp.where(mask, phi, 0.0)
    feats = jnp.stack([rho, theta / jnp.pi, phi / jnp.pi,
                       jnp.zeros_like(rho)], axis=1)

    W0p = jnp.concatenate([0.1 * Wk0[:d], jnp.zeros((1, h), jnp.float32)],
                          axis=0)
    b0 = (0.1 * bk0)[None, :]
    y = z[src]
    se = s[src]

    msg = _edge_stage(feats, y, se, W0p, b0, cSg, sSg, W1d, b1d, eb=2000)

    agg = jax.ops.segment_sum(msg, dst, num_segments=n)

    return _final_stage(agg, z, s, H1self, conv_bias, W_dec1, b_dec1, nb=2000)
